# baseline (speedup 1.0000x reference)
import math

import jax
import jax.numpy as jnp
from jax.experimental import pallas as pl

H = 1024
NH = 16
HD = H // NH
W = 128
TOPK = 64
KSEL = TOPK - 2
T = 2048
NB = T // W

def _dot(a, b, dims):
    return jax.lax.dot_general(a, b, (dims, ((), ())),
                               preferred_element_type=jnp.float32)


def _dotx(a, b, dims):
    return jax.lax.dot_general(a, b, (dims, ((), ())),
                               preferred_element_type=jnp.float32,
                               precision=jax.lax.Precision.HIGHEST)


def _softmax_rows(s):
    m = jnp.max(s, axis=-1, keepdims=True)
    e = jnp.exp(s - m)
    return e / jnp.sum(e, axis=-1, keepdims=True)


def _excl_prefix(x):
    rio = jax.lax.broadcasted_iota(jnp.int32, (W, W), 0)
    cio = jax.lax.broadcasted_iota(jnp.int32, (W, W), 1)
    upper = (rio <= cio).astype(jnp.float32)
    incl = _dot(x, upper, (((1,), (0,))))
    rt = incl[:, W - 1:W]
    a = jax.lax.broadcasted_iota(jnp.int32, (NB, NB), 0)
    b = jax.lax.broadcasted_iota(jnp.int32, (NB, NB), 1)
    lower = (b < a).astype(jnp.float32)
    offs = _dot(lower, rt, (((1,), (0,))))
    return incl - x + offs


def _proj_kernel(x_ref, w_ref, b_ref, out_ref):
    x = x_ref[0]
    w = w_ref[0]
    y = _dot(x, w, (((1,), (0,)))) + b_ref[0]
    for h in range(NH):
        out_ref[0, 0, h] = y[:, h * HD:(h + 1) * HD]


def _attn_kernel(qkv_ref, out_ref):
    scale = 1.0 / math.sqrt(HD)
    for h in range(2):
        q = qkv_ref[0, 0, h]
        k = qkv_ref[1, 0, h]
        v = qkv_ref[2, 0, h]

        q3 = q.reshape(NB, W, HD)
        ns = jnp.sum(q3 * q3, axis=2)
        bits = jax.lax.bitcast_convert_type(ns, jnp.int32)

        def bit_body(i, t):
            cand = t | (jnp.int32(1) << (jnp.int32(30) - i))
            cnt = jnp.sum((bits >= cand).astype(jnp.int32))
            return jnp.where(cnt >= KSEL, cand, t)

        thr = jax.lax.fori_loop(0, 31, bit_body, jnp.int32(0))
        gt = bits > thr
        tie = bits == thr
        need = (KSEL - jnp.sum(gt.astype(jnp.int32))).astype(jnp.float32)
        tie_rank = _excl_prefix(tie.astype(jnp.float32))
        sel = gt | (tie & (tie_rank < need))
        rio = jax.lax.broadcasted_iota(jnp.int32, (NB, W), 0)
        cio = jax.lax.broadcasted_iota(jnp.int32, (NB, W), 1)
        flat = rio * W + cio
        m = sel | (flat == 0) | (flat == T - 1)
        mf = m.astype(jnp.float32)

        em = _excl_prefix(mf).astype(jnp.int32)
        r64 = jax.lax.broadcasted_iota(jnp.int32, (TOPK, NB, W), 0)
        p3 = jnp.where((r64 == em[None]) & m[None], 1.0, 0.0)

        qg = jnp.zeros((TOPK, HD), jnp.float32)
        for b in range(NB):
            qg = qg + _dotx(p3[:, b, :], q3[b], (((1,), (0,))))
        gs = _dot(qg, k, (((1,), (1,)))) * scale
        gp = _softmax_rows(gs)
        gctx = _dot(gp, v, (((1,), (0,))))

        ones_t = jnp.ones((TOPK, 1), jnp.float32)

        for b in range(NB):
            lo = max(0, (b - 1) * W)
            hi = min(T, (b + 2) * W)
            qb = q[b * W:(b + 1) * W]
            kw = k[lo:hi]
            vw = v[lo:hi]
            s = _dot(qb, kw, (((1,), (1,)))) * scale
            p = _softmax_rows(s)
            lb = _dot(p, vw, (((1,), (0,))))
            pb = p3[:, b, :]
            mcol = _dot(pb, ones_t, (((0,), (0,))))
            scat = _dotx(pb, gctx, (((0,), (0,))))
            out_ref[0, b * W:(b + 1) * W, h * HD:(h + 1) * HD] = (
                lb * (1.0 - mcol) + scat)


def kernel(hidden_states, Wq, bq, Wk, bk, Wv, bv):
    n, t, _ = hidden_states.shape
    ws = jnp.stack([Wq, Wk, Wv])
    bs = jnp.stack([bq, bk, bv]).reshape(3, 1, H)
    nt = t // W

    qkv = pl.pallas_call(
        _proj_kernel,
        grid=(3, n, nt),
        in_specs=[
            pl.BlockSpec((1, W, H), lambda p, ni, ti: (ni, ti, 0)),
            pl.BlockSpec((1, H, H), lambda p, ni, ti: (p, 0, 0)),
            pl.BlockSpec((1, 1, H), lambda p, ni, ti: (p, 0, 0)),
        ],
        out_specs=pl.BlockSpec((1, 1, NH, W, HD),
                               lambda p, ni, ti: (p, ni, 0, ti, 0)),
        out_shape=jax.ShapeDtypeStruct((3, n, NH, t, HD), jnp.float32),
    )(hidden_states, ws, bs)

    out = pl.pallas_call(
        _attn_kernel,
        grid=(n, NH // 2),
        in_specs=[pl.BlockSpec((3, 1, 2, t, HD), lambda ni, hp: (0, ni, hp, 0, 0))],
        out_specs=pl.BlockSpec((1, t, 2 * HD), lambda ni, hp: (ni, 0, hp)),
        out_shape=jax.ShapeDtypeStruct((n, t, H), jnp.float32),
    )(qkv)
    return out


# speedup vs baseline: 1.1966x; 1.1966x over previous
import math

import jax
import jax.numpy as jnp
from jax.experimental import pallas as pl
from jax.experimental.pallas import tpu as pltpu

H = 1024
NH = 16
HD = H // NH
W = 128
TOPK = 64
KSEL = TOPK - 2
T = 2048
NB = T // W


def _dot(a, b, dims):
    return jax.lax.dot_general(a, b, (dims, ((), ())),
                               preferred_element_type=jnp.float32)


def _dotx(a, b, dims):
    return jax.lax.dot_general(a, b, (dims, ((), ())),
                               preferred_element_type=jnp.float32,
                               precision=jax.lax.Precision.HIGHEST)


def _softmax_rows(s):
    m = jnp.max(s, axis=-1, keepdims=True)
    e = jnp.exp(s - m)
    return e / jnp.sum(e, axis=-1, keepdims=True)


def _excl_prefix(x):
    rio = jax.lax.broadcasted_iota(jnp.int32, (W, W), 0)
    cio = jax.lax.broadcasted_iota(jnp.int32, (W, W), 1)
    upper = (rio <= cio).astype(jnp.float32)
    incl = _dot(x, upper, (((1,), (0,))))
    rt = incl[:, W - 1:W]
    a = jax.lax.broadcasted_iota(jnp.int32, (NB, NB), 0)
    b = jax.lax.broadcasted_iota(jnp.int32, (NB, NB), 1)
    lower = (b < a).astype(jnp.float32)
    offs = _dot(lower, rt, (((1,), (0,))))
    return incl - x + offs


def _proj_kernel(x_ref, wq_ref, bq_ref, wk_ref, bk_ref, wv_ref, bv_ref,
                 out_ref):
    x = x_ref[0]
    for p, (w_ref, b_ref) in enumerate(((wq_ref, bq_ref), (wk_ref, bk_ref),
                                        (wv_ref, bv_ref))):
        y = _dot(x, w_ref[...], (((1,), (0,)))) + b_ref[...]
        for h in range(NH):
            out_ref[p, 0, h] = y[:, h * HD:(h + 1) * HD]


def _attn_kernel(qkv_ref, out_ref, p2d_ref):
    scale = 1.0 / math.sqrt(HD)
    for h in range(2):
        q = qkv_ref[0, 0, h]
        k16 = qkv_ref[1, 0, h].astype(jnp.bfloat16)
        v16 = qkv_ref[2, 0, h].astype(jnp.bfloat16)
        q16 = q.astype(jnp.bfloat16)

        q3 = q.reshape(NB, W, HD)
        ns = jnp.sum(q3 * q3, axis=2)
        bits = jax.lax.bitcast_convert_type(ns, jnp.int32)

        def bit_body(i, t):
            cand = t | (jnp.int32(1) << (jnp.int32(30) - i))
            cnt = jnp.sum((bits >= cand).astype(jnp.int32))
            return jnp.where(cnt >= KSEL, cand, t)

        thr = jax.lax.fori_loop(0, 31, bit_body, jnp.int32(0))
        gt = bits > thr
        tie = bits == thr
        need = (KSEL - jnp.sum(gt.astype(jnp.int32))).astype(jnp.float32)
        tie_rank = _excl_prefix(tie.astype(jnp.float32))
        sel = gt | (tie & (tie_rank < need))
        rio = jax.lax.broadcasted_iota(jnp.int32, (NB, W), 0)
        cio = jax.lax.broadcasted_iota(jnp.int32, (NB, W), 1)
        flat = rio * W + cio
        m = sel | (flat == 0) | (flat == T - 1)
        mf = m.astype(jnp.float32)

        em = _excl_prefix(mf).astype(jnp.int32)
        r64 = jax.lax.broadcasted_iota(jnp.int32, (TOPK, NB, W), 0)
        p3 = jnp.where((r64 == em[None]) & m[None], 1.0, 0.0)
        for b in range(NB):
            p2d_ref[:, b * W:(b + 1) * W] = p3[:, b, :]
        p2 = p2d_ref[...]

        qg = _dot(p2, q, (((1,), (0,))))
        gs = _dot(qg.astype(jnp.bfloat16), k16, (((1,), (1,)))) * scale
        gp = _softmax_rows(gs)
        gctx = _dot(gp.astype(jnp.bfloat16), v16, (((1,), (0,))))

        mcol = _dot(p2, jnp.ones((TOPK, 1), jnp.float32), (((0,), (0,))))
        scat = _dotx(p2, gctx, (((0,), (0,))))

        for b in range(NB):
            lo = max(0, (b - 1) * W)
            hi = min(T, (b + 2) * W)
            qb = q16[b * W:(b + 1) * W]
            s = _dot(qb, k16[lo:hi], (((1,), (1,)))) * scale
            p = _softmax_rows(s)
            lb = _dot(p.astype(jnp.bfloat16), v16[lo:hi], (((1,), (0,))))
            sl = slice(b * W, (b + 1) * W)
            out_ref[0, sl, h * HD:(h + 1) * HD] = (
                lb * (1.0 - mcol[sl]) + scat[sl])


def kernel(hidden_states, Wq, bq, Wk, bk, Wv, bv):
    n, t, _ = hidden_states.shape
    nt = t // W
    wspec = pl.BlockSpec((H, H), lambda ni, ti: (0, 0))
    bspec = pl.BlockSpec((1, H), lambda ni, ti: (0, 0))

    qkv = pl.pallas_call(
        _proj_kernel,
        grid=(n, nt),
        in_specs=[
            pl.BlockSpec((1, W, H), lambda ni, ti: (ni, ti, 0)),
            wspec, bspec, wspec, bspec, wspec, bspec,
        ],
        out_specs=pl.BlockSpec((3, 1, NH, W, HD),
                               lambda ni, ti: (0, ni, 0, ti, 0)),
        out_shape=jax.ShapeDtypeStruct((3, n, NH, t, HD), jnp.float32),
    )(hidden_states, Wq, bq.reshape(1, H), Wk, bk.reshape(1, H),
      Wv, bv.reshape(1, H))

    out = pl.pallas_call(
        _attn_kernel,
        grid=(n, NH // 2),
        in_specs=[pl.BlockSpec((3, 1, 2, t, HD),
                               lambda ni, hp: (0, ni, hp, 0, 0))],
        out_specs=pl.BlockSpec((1, t, 2 * HD), lambda ni, hp: (ni, 0, hp)),
        out_shape=jax.ShapeDtypeStruct((n, t, H), jnp.float32),
        scratch_shapes=[pltpu.VMEM((TOPK, T), jnp.float32)],
    )(qkv)
    return out
